# Initial kernel scaffold; baseline (speedup 1.0000x reference)
import jax
import jax.numpy as jnp
from jax.experimental import pallas as pl


def kernel(hist_iid_seq, hist_aid_seq, hist_rate_seq, hist_seq_len, iid, aid, lb, item_table, attr_table, rating_table):
    raise NotImplementedError("write your pallas kernel here")


# speedup vs baseline: 8.7165x; 8.7165x over previous
import functools

import jax
import jax.numpy as jnp
from jax import lax
from jax.experimental import pallas as pl
from jax.experimental.pallas import tpu as pltpu
from jax.experimental.pallas import tpu_sc as plsc

B = 4096
L = 200
E = 32
ATTR_FNUM = 2
NRATE = 5
NW = 32
BPW = B // NW
H = E // 2
INV_L = 1.0 / L
NKV = 13


def _body(hii, hai, hri, tii, tai, item_t, attr_t, rate_t, out,
          idx_i, idx_a, rate_buf, rows_i, rows_a, rtab,
          ti_idx, ta_idx, rows_ti, rows_ta, out_v, sem):
    wid = lax.axis_index("s") * 2 + lax.axis_index("c")
    base = wid * BPW

    zf = jnp.zeros((16,), jnp.float32)

    pltpu.sync_copy(rate_t, rtab)
    rtab[NRATE, pl.ds(0, H)] = zf
    rtab[NRATE, pl.ds(H, H)] = zf

    pltpu.sync_copy(tii.at[wid], ti_idx)
    pltpu.sync_copy(tai.at[wid], ta_idx)
    cps = [pltpu.async_copy(item_t.at[ti_idx], rows_ti, sem),
           pltpu.async_copy(attr_t.at[ta_idx.at[0]], rows_ta.at[0], sem),
           pltpu.async_copy(attr_t.at[ta_idx.at[1]], rows_ta.at[1], sem)]
    for c in cps:
        c.wait()

    def tgt_item_body(r, _):
        out_v[r, 0, pl.ds(0, H)] = rows_ti[r, pl.ds(0, H)]
        out_v[r, 0, pl.ds(H, H)] = rows_ti[r, pl.ds(H, H)]
        return 0
    lax.fori_loop(0, BPW, tgt_item_body, 0)

    for h in range(2):
        def tgt_attr_body(q, _, h=h):
            r = 64 * h + q
            for s in range(2):
                j = 2 * q + s
                out_v[r, 1 + s, pl.ds(0, H)] = rows_ta[h, j, pl.ds(0, H)]
                out_v[r, 1 + s, pl.ds(H, H)] = rows_ta[h, j, pl.ds(H, H)]
            return 0
        lax.fori_loop(0, 64, tgt_attr_body, 0)

    def row_body(r, _):
        row = base + r
        pltpu.sync_copy(hii.at[row], idx_i)
        pltpu.sync_copy(hai.at[row], idx_a)
        pltpu.sync_copy(hri.at[row], rate_buf)
        g = [pltpu.async_copy(item_t.at[idx_i.at[c]], rows_i.at[c], sem)
             for c in range(2)]
        g += [pltpu.async_copy(attr_t.at[idx_a.at[c]], rows_a.at[c], sem)
              for c in range(4)]
        for c in g:
            c.wait()

        acc = (zf, zf)
        for c in range(2):
            def ibody(j, a, c=c):
                return (a[0] + rows_i[c, j, pl.ds(0, H)],
                        a[1] + rows_i[c, j, pl.ds(H, H)])
            acc = lax.fori_loop(0, 100, ibody, acc)
        out_v[r, 3, pl.ds(0, H)] = acc[0] * INV_L
        out_v[r, 3, pl.ds(H, H)] = acc[1] * INV_L

        a45 = ((zf, zf), (zf, zf))
        for c in range(4):
            def abody(j, a, c=c):
                (l4, h4), (l5, h5) = a
                return ((l4 + rows_a[c, 2 * j, pl.ds(0, H)],
                         h4 + rows_a[c, 2 * j, pl.ds(H, H)]),
                        (l5 + rows_a[c, 2 * j + 1, pl.ds(0, H)],
                         h5 + rows_a[c, 2 * j + 1, pl.ds(H, H)]))
            a45 = lax.fori_loop(0, 50, abody, a45)
        out_v[r, 4, pl.ds(0, H)] = a45[0][0] * INV_L
        out_v[r, 4, pl.ds(H, H)] = a45[0][1] * INV_L
        out_v[r, 5, pl.ds(0, H)] = a45[1][0] * INV_L
        out_v[r, 5, pl.ds(H, H)] = a45[1][1] * INV_L

        def rbody(k, a):
            v = rate_buf[k, :]
            lo, hi = a
            for i in range(16):
                rl = v[i]
                lo = lo + rtab[rl, pl.ds(0, H)]
                hi = hi + rtab[rl, pl.ds(H, H)]
            return (lo, hi)
        lo, hi = lax.fori_loop(0, NKV, rbody, (zf, zf))
        out_v[r, 6, pl.ds(0, H)] = lo * INV_L
        out_v[r, 6, pl.ds(H, H)] = hi * INV_L
        return 0

    lax.fori_loop(0, BPW, row_body, 0)
    pltpu.sync_copy(out_v, out.at[pl.ds(base, BPW)])


_sc_call = functools.partial(
    pl.kernel,
    out_type=jax.ShapeDtypeStruct((B, 7, E), jnp.float32),
    mesh=plsc.VectorSubcoreMesh(core_axis_name="c", subcore_axis_name="s"),
    compiler_params=pltpu.CompilerParams(use_tc_tiling_on_sc=False),
    scratch_types=[
        pltpu.VMEM((2, 100), jnp.int32),
        pltpu.VMEM((4, 100), jnp.int32),
        pltpu.VMEM((NKV, 16), jnp.int32),
        pltpu.VMEM((2, 100, E), jnp.float32),
        pltpu.VMEM((4, 100, E), jnp.float32),
        pltpu.VMEM((NRATE + 1, E), jnp.float32),
        pltpu.VMEM((BPW,), jnp.int32),
        pltpu.VMEM((2, BPW), jnp.int32),
        pltpu.VMEM((BPW, E), jnp.float32),
        pltpu.VMEM((2, BPW, E), jnp.float32),
        pltpu.VMEM((BPW, 7, E), jnp.float32),
        pltpu.SemaphoreType.DMA,
    ],
)(_body)


def kernel(hist_iid_seq, hist_aid_seq, hist_rate_seq, hist_seq_len, iid, aid,
           lb, item_table, attr_table, rating_table):
    hii = hist_iid_seq.astype(jnp.int32).reshape(B, 2, 100)
    hai = hist_aid_seq.astype(jnp.int32).reshape(B, 4, 100)
    hri = jnp.pad(hist_rate_seq.astype(jnp.int32), ((0, 0), (0, 8)),
                  constant_values=NRATE).reshape(B, NKV, 16)
    tii = iid.astype(jnp.int32).reshape(NW, BPW)
    tai = aid.astype(jnp.int32).reshape(NW, 2, BPW)
    return _sc_call(hii, hai, hri, tii, tai,
                    item_table, attr_table, rating_table)


# speedup vs baseline: 13.4808x; 1.5466x over previous
import functools

import jax
import jax.numpy as jnp
from jax import lax
from jax.experimental import pallas as pl
from jax.experimental.pallas import tpu as pltpu
from jax.experimental.pallas import tpu_sc as plsc

B = 4096
L = 200
E = 32
ATTR_FNUM = 2
NRATE = 5
NW = 32
BPW = B // NW
H = E // 2
INV_L = 1.0 / L
NKV = 13


def _body(hii, hai, hri, tii, tai, item_t, attr_t, rate_t, out,
          idx_i, idx_a, rate_buf, rows_i, rows_a, rtab,
          ti_idx, ta_idx, rows_ti, rows_ta, out_v, sem,
          sem_i0, sem_i1, sem_r0, sem_r1, sem_g0, sem_g1):
    wid = lax.axis_index("s") * 2 + lax.axis_index("c")
    base = wid * BPW
    zf = jnp.zeros((16,), jnp.float32)

    sem_i = (sem_i0, sem_i1)
    sem_r = (sem_r0, sem_r1)
    sem_g = (sem_g0, sem_g1)

    def issue_idx(row, p):
        pltpu.async_copy(hii.at[row], idx_i.at[p], sem_i[p])
        pltpu.async_copy(hai.at[row], idx_a.at[p], sem_i[p])

    def wait_idx(row, p):
        pltpu.make_async_copy(hii.at[row], idx_i.at[p], sem_i[p]).wait()
        pltpu.make_async_copy(hai.at[row], idx_a.at[p], sem_i[p]).wait()

    def issue_rate(row, p):
        pltpu.async_copy(hri.at[row], rate_buf.at[p], sem_r[p])

    def wait_rate(row, p):
        pltpu.make_async_copy(hri.at[row], rate_buf.at[p], sem_r[p]).wait()

    def issue_gathers(p):
        for c in range(2):
            pltpu.async_copy(item_t.at[idx_i.at[p, c]], rows_i.at[p, c],
                             sem_g[p])
        for c in range(4):
            pltpu.async_copy(attr_t.at[idx_a.at[p, c]], rows_a.at[p, c],
                             sem_g[p])

    def wait_gathers(p):
        for c in range(2):
            pltpu.make_async_copy(item_t.at[idx_i.at[p, c]],
                                  rows_i.at[p, c], sem_g[p]).wait()
        for c in range(4):
            pltpu.make_async_copy(attr_t.at[idx_a.at[p, c]],
                                  rows_a.at[p, c], sem_g[p]).wait()

    issue_idx(base + 0, 0)
    issue_idx(base + 1, 1)
    issue_rate(base + 0, 0)
    issue_rate(base + 1, 1)

    pltpu.sync_copy(rate_t, rtab)
    rtab[NRATE, pl.ds(0, H)] = zf
    rtab[NRATE, pl.ds(H, H)] = zf

    pltpu.sync_copy(tii.at[wid], ti_idx)
    pltpu.sync_copy(tai.at[wid], ta_idx)
    cps = [pltpu.async_copy(item_t.at[ti_idx], rows_ti, sem),
           pltpu.async_copy(attr_t.at[ta_idx.at[0]], rows_ta.at[0], sem),
           pltpu.async_copy(attr_t.at[ta_idx.at[1]], rows_ta.at[1], sem)]
    for c in cps:
        c.wait()

    def tgt_item_body(r, _):
        out_v[r, 0, pl.ds(0, H)] = rows_ti[r, pl.ds(0, H)]
        out_v[r, 0, pl.ds(H, H)] = rows_ti[r, pl.ds(H, H)]
        return 0
    lax.fori_loop(0, BPW, tgt_item_body, 0)

    for h in range(2):
        def tgt_attr_body(q, _, h=h):
            r = 64 * h + q
            for s in range(2):
                j = 2 * q + s
                out_v[r, 1 + s, pl.ds(0, H)] = rows_ta[h, j, pl.ds(0, H)]
                out_v[r, 1 + s, pl.ds(H, H)] = rows_ta[h, j, pl.ds(H, H)]
            return 0
        lax.fori_loop(0, 64, tgt_attr_body, 0)

    wait_idx(base + 0, 0)
    issue_gathers(0)
    wait_idx(base + 1, 1)
    issue_gathers(1)

    def accum(r, p):
        acc = (zf, zf)
        for c in range(2):
            def ibody(j, a, c=c):
                return (a[0] + rows_i[p, c, j, pl.ds(0, H)],
                        a[1] + rows_i[p, c, j, pl.ds(H, H)])
            acc = lax.fori_loop(0, 100, ibody, acc, unroll=4)
        out_v[r, 3, pl.ds(0, H)] = acc[0] * INV_L
        out_v[r, 3, pl.ds(H, H)] = acc[1] * INV_L

        a45 = ((zf, zf), (zf, zf))
        for c in range(4):
            def abody(j, a, c=c):
                (l4, h4), (l5, h5) = a
                return ((l4 + rows_a[p, c, 2 * j, pl.ds(0, H)],
                         h4 + rows_a[p, c, 2 * j, pl.ds(H, H)]),
                        (l5 + rows_a[p, c, 2 * j + 1, pl.ds(0, H)],
                         h5 + rows_a[p, c, 2 * j + 1, pl.ds(H, H)]))
            a45 = lax.fori_loop(0, 50, abody, a45, unroll=4)
        out_v[r, 4, pl.ds(0, H)] = a45[0][0] * INV_L
        out_v[r, 4, pl.ds(H, H)] = a45[0][1] * INV_L
        out_v[r, 5, pl.ds(0, H)] = a45[1][0] * INV_L
        out_v[r, 5, pl.ds(H, H)] = a45[1][1] * INV_L

        def rbody(k, a):
            v = rate_buf[p, k, :]
            lo, hi = a
            for i in range(16):
                rl = v[i]
                lo = lo + rtab[rl, pl.ds(0, H)]
                hi = hi + rtab[rl, pl.ds(H, H)]
            return (lo, hi)
        lo, hi = lax.fori_loop(0, NKV, rbody, (zf, zf))
        out_v[r, 6, pl.ds(0, H)] = lo * INV_L
        out_v[r, 6, pl.ds(H, H)] = hi * INV_L

    def gbody(g, _):
        for p in range(2):
            r = 2 * g + p
            wait_gathers(p)

            @pl.when(r + 2 < BPW)
            def _():
                issue_idx(base + r + 2, p)

            wait_rate(base + r, p)
            accum(r, p)

            @pl.when(r + 2 < BPW)
            def _():
                issue_rate(base + r + 2, p)
                wait_idx(base + r + 2, p)
                issue_gathers(p)
        return 0

    lax.fori_loop(0, BPW // 2, gbody, 0)
    pltpu.sync_copy(out_v, out.at[pl.ds(base, BPW)])


_sc_call = functools.partial(
    pl.kernel,
    out_type=jax.ShapeDtypeStruct((B, 7, E), jnp.float32),
    mesh=plsc.VectorSubcoreMesh(core_axis_name="c", subcore_axis_name="s"),
    compiler_params=pltpu.CompilerParams(use_tc_tiling_on_sc=False),
    scratch_types=[
        pltpu.VMEM((2, 2, 100), jnp.int32),
        pltpu.VMEM((2, 4, 100), jnp.int32),
        pltpu.VMEM((2, NKV, 16), jnp.int32),
        pltpu.VMEM((2, 2, 100, E), jnp.float32),
        pltpu.VMEM((2, 4, 100, E), jnp.float32),
        pltpu.VMEM((NRATE + 1, E), jnp.float32),
        pltpu.VMEM((BPW,), jnp.int32),
        pltpu.VMEM((2, BPW), jnp.int32),
        pltpu.VMEM((BPW, E), jnp.float32),
        pltpu.VMEM((2, BPW, E), jnp.float32),
        pltpu.VMEM((BPW, 7, E), jnp.float32),
        pltpu.SemaphoreType.DMA,
        pltpu.SemaphoreType.DMA,
        pltpu.SemaphoreType.DMA,
        pltpu.SemaphoreType.DMA,
        pltpu.SemaphoreType.DMA,
        pltpu.SemaphoreType.DMA,
        pltpu.SemaphoreType.DMA,
    ],
)(_body)


def kernel(hist_iid_seq, hist_aid_seq, hist_rate_seq, hist_seq_len, iid, aid,
           lb, item_table, attr_table, rating_table):
    hii = hist_iid_seq.astype(jnp.int32).reshape(B, 2, 100)
    hai = hist_aid_seq.astype(jnp.int32).reshape(B, 4, 100)
    hri = jnp.pad(hist_rate_seq.astype(jnp.int32), ((0, 0), (0, 8)),
                  constant_values=NRATE).reshape(B, NKV, 16)
    tii = iid.astype(jnp.int32).reshape(NW, BPW)
    tai = aid.astype(jnp.int32).reshape(NW, 2, BPW)
    return _sc_call(hii, hai, hri, tii, tai,
                    item_table, attr_table, rating_table)


# speedup vs baseline: 13.6565x; 1.0130x over previous
import functools

import jax
import jax.numpy as jnp
from jax import lax
from jax.experimental import pallas as pl
from jax.experimental.pallas import tpu as pltpu
from jax.experimental.pallas import tpu_sc as plsc

B = 4096
L = 200
E = 32
ATTR_FNUM = 2
NRATE = 5
NW = 32
BPW = B // NW
H = E // 2
INV_L = 1.0 / L
NKV = 13


def _body(hii, hai, hri, tii, tai, item_t, attr_t, rate_t, out,
          idx_i, idx_a, rate_buf, rows_i, rows_a, rtab,
          ti_idx, ta_idx, rows_ti, rows_ta, out_v, sem,
          sem_i0, sem_i1, sem_r0, sem_r1, sem_g0, sem_g1):
    wid = lax.axis_index("s") * 2 + lax.axis_index("c")
    base = wid * BPW
    zf = jnp.zeros((16,), jnp.float32)

    sem_i = (sem_i0, sem_i1)
    sem_r = (sem_r0, sem_r1)
    sem_g = (sem_g0, sem_g1)

    def issue_idx(row, p):
        pltpu.async_copy(hii.at[row], idx_i.at[p], sem_i[p])
        pltpu.async_copy(hai.at[row], idx_a.at[p], sem_i[p])

    def wait_idx(row, p):
        pltpu.make_async_copy(hii.at[row], idx_i.at[p], sem_i[p]).wait()
        pltpu.make_async_copy(hai.at[row], idx_a.at[p], sem_i[p]).wait()

    def issue_rate(row, p):
        pltpu.async_copy(hri.at[row], rate_buf.at[p], sem_r[p])

    def wait_rate(row, p):
        pltpu.make_async_copy(hri.at[row], rate_buf.at[p], sem_r[p]).wait()

    def issue_gathers(p):
        for c in range(2):
            pltpu.async_copy(item_t.at[idx_i.at[p, c]], rows_i.at[p, c],
                             sem_g[p])
        for c in range(4):
            pltpu.async_copy(attr_t.at[idx_a.at[p, c]], rows_a.at[p, c],
                             sem_g[p])

    def wait_gathers(p):
        for c in range(2):
            pltpu.make_async_copy(item_t.at[idx_i.at[p, c]],
                                  rows_i.at[p, c], sem_g[p]).wait()
        for c in range(4):
            pltpu.make_async_copy(attr_t.at[idx_a.at[p, c]],
                                  rows_a.at[p, c], sem_g[p]).wait()

    issue_idx(base + 0, 0)
    issue_idx(base + 1, 1)
    issue_rate(base + 0, 0)
    issue_rate(base + 1, 1)

    pltpu.sync_copy(rate_t, rtab)
    rt = [(rtab[rr, pl.ds(0, H)], rtab[rr, pl.ds(H, H)])
          for rr in range(NRATE)]

    pltpu.sync_copy(tii.at[wid], ti_idx)
    pltpu.sync_copy(tai.at[wid], ta_idx)
    cps = [pltpu.async_copy(item_t.at[ti_idx], rows_ti, sem),
           pltpu.async_copy(attr_t.at[ta_idx.at[0]], rows_ta.at[0], sem),
           pltpu.async_copy(attr_t.at[ta_idx.at[1]], rows_ta.at[1], sem)]
    for c in cps:
        c.wait()

    def tgt_item_body(r, _):
        out_v[r, 0, pl.ds(0, H)] = rows_ti[r, pl.ds(0, H)]
        out_v[r, 0, pl.ds(H, H)] = rows_ti[r, pl.ds(H, H)]
        return 0
    lax.fori_loop(0, BPW, tgt_item_body, 0)

    for h in range(2):
        def tgt_attr_body(q, _, h=h):
            r = 64 * h + q
            for s in range(2):
                j = 2 * q + s
                out_v[r, 1 + s, pl.ds(0, H)] = rows_ta[h, j, pl.ds(0, H)]
                out_v[r, 1 + s, pl.ds(H, H)] = rows_ta[h, j, pl.ds(H, H)]
            return 0
        lax.fori_loop(0, 64, tgt_attr_body, 0)

    wait_idx(base + 0, 0)
    issue_gathers(0)
    wait_idx(base + 1, 1)
    issue_gathers(1)

    def accum(r, p):
        acc = (zf, zf)
        for c in range(2):
            def ibody(j, a, c=c):
                return (a[0] + rows_i[p, c, j, pl.ds(0, H)],
                        a[1] + rows_i[p, c, j, pl.ds(H, H)])
            acc = lax.fori_loop(0, 100, ibody, acc, unroll=4)
        out_v[r, 3, pl.ds(0, H)] = acc[0] * INV_L
        out_v[r, 3, pl.ds(H, H)] = acc[1] * INV_L

        a45 = ((zf, zf), (zf, zf))
        for c in range(4):
            def abody(j, a, c=c):
                (l4, h4), (l5, h5) = a
                return ((l4 + rows_a[p, c, 2 * j, pl.ds(0, H)],
                         h4 + rows_a[p, c, 2 * j, pl.ds(H, H)]),
                        (l5 + rows_a[p, c, 2 * j + 1, pl.ds(0, H)],
                         h5 + rows_a[p, c, 2 * j + 1, pl.ds(H, H)]))
            a45 = lax.fori_loop(0, 50, abody, a45, unroll=4)
        out_v[r, 4, pl.ds(0, H)] = a45[0][0] * INV_L
        out_v[r, 4, pl.ds(H, H)] = a45[0][1] * INV_L
        out_v[r, 5, pl.ds(0, H)] = a45[1][0] * INV_L
        out_v[r, 5, pl.ds(H, H)] = a45[1][1] * INV_L

        onef = jnp.ones((16,), jnp.float32)

        def rbody(k, a):
            v = rate_buf[p, k, :]
            return [a[rr] + jnp.where(v == rr, onef, zf)
                    for rr in range(NRATE)]
        cnts = lax.fori_loop(0, NKV, rbody, [zf] * NRATE)
        lane = jnp.arange(16, dtype=jnp.int32)
        rots = [((lane + sh) & 15) for sh in (8, 4, 2, 1)]
        dnums = lax.GatherDimensionNumbers(
            offset_dims=(), collapsed_slice_dims=(0,), start_index_map=(0,))

        def rot16(c, rot):
            return lax.gather(c, rot[:, None], dnums, (1,),
                              mode=lax.GatherScatterMode.PROMISE_IN_BOUNDS)

        lo, hi = zf, zf
        for rr in range(NRATE):
            c = cnts[rr]
            for rot in rots:
                c = c + rot16(c, rot)
            lo = lo + c * rt[rr][0]
            hi = hi + c * rt[rr][1]
        out_v[r, 6, pl.ds(0, H)] = lo * INV_L
        out_v[r, 6, pl.ds(H, H)] = hi * INV_L

    def gbody(g, _):
        for p in range(2):
            r = 2 * g + p
            wait_gathers(p)

            @pl.when(r + 2 < BPW)
            def _():
                issue_idx(base + r + 2, p)

            wait_rate(base + r, p)
            accum(r, p)

            @pl.when(r + 2 < BPW)
            def _():
                issue_rate(base + r + 2, p)
                wait_idx(base + r + 2, p)
                issue_gathers(p)
        return 0

    lax.fori_loop(0, BPW // 2, gbody, 0)
    pltpu.sync_copy(out_v, out.at[pl.ds(base, BPW)])


_sc_call = functools.partial(
    pl.kernel,
    out_type=jax.ShapeDtypeStruct((B, 7, E), jnp.float32),
    mesh=plsc.VectorSubcoreMesh(core_axis_name="c", subcore_axis_name="s"),
    compiler_params=pltpu.CompilerParams(use_tc_tiling_on_sc=False),
    scratch_types=[
        pltpu.VMEM((2, 2, 100), jnp.int32),
        pltpu.VMEM((2, 4, 100), jnp.int32),
        pltpu.VMEM((2, NKV, 16), jnp.int32),
        pltpu.VMEM((2, 2, 100, E), jnp.float32),
        pltpu.VMEM((2, 4, 100, E), jnp.float32),
        pltpu.VMEM((NRATE + 1, E), jnp.float32),
        pltpu.VMEM((BPW,), jnp.int32),
        pltpu.VMEM((2, BPW), jnp.int32),
        pltpu.VMEM((BPW, E), jnp.float32),
        pltpu.VMEM((2, BPW, E), jnp.float32),
        pltpu.VMEM((BPW, 7, E), jnp.float32),
        pltpu.SemaphoreType.DMA,
        pltpu.SemaphoreType.DMA,
        pltpu.SemaphoreType.DMA,
        pltpu.SemaphoreType.DMA,
        pltpu.SemaphoreType.DMA,
        pltpu.SemaphoreType.DMA,
        pltpu.SemaphoreType.DMA,
    ],
)(_body)


def kernel(hist_iid_seq, hist_aid_seq, hist_rate_seq, hist_seq_len, iid, aid,
           lb, item_table, attr_table, rating_table):
    hii = hist_iid_seq.astype(jnp.int32).reshape(B, 2, 100)
    hai = hist_aid_seq.astype(jnp.int32).reshape(B, 4, 100)
    hri = jnp.pad(hist_rate_seq.astype(jnp.int32), ((0, 0), (0, 8)),
                  constant_values=NRATE).reshape(B, NKV, 16)
    tii = iid.astype(jnp.int32).reshape(NW, BPW)
    tai = aid.astype(jnp.int32).reshape(NW, 2, BPW)
    return _sc_call(hii, hai, hri, tii, tai,
                    item_table, attr_table, rating_table)


# speedup vs baseline: 14.0585x; 1.0294x over previous
import functools

import jax
import jax.numpy as jnp
from jax import lax
from jax.experimental import pallas as pl
from jax.experimental.pallas import tpu as pltpu
from jax.experimental.pallas import tpu_sc as plsc

B = 4096
L = 200
E = 32
NRATE = 5
NW = 32
BPW = B // NW
H = E // 2
INV_L = 1.0 / L
NKV = 13

I_CH = ((0, 104), (104, 96))
A_CH = ((0, 104), (104, 104), (208, 104), (312, 88))


def _body(hii, hai, hri, tii, tai, item_t, attr_t, rate_t, out,
          idx_i, idx_a, rate_buf, rows_i, rows_a, rtab,
          ti_idx, ta_idx, rows_ti, rows_ta, out_v, sem,
          sem_i0, sem_i1, sem_r0, sem_r1, sem_g0, sem_g1):
    wid = lax.axis_index("s") * 2 + lax.axis_index("c")
    base = wid * BPW
    zf = jnp.zeros((16,), jnp.float32)

    sem_i = (sem_i0, sem_i1)
    sem_r = (sem_r0, sem_r1)
    sem_g = (sem_g0, sem_g1)

    def issue_idx(row, p):
        pltpu.async_copy(hii.at[pl.ds(row, 1), :], idx_i.at[p], sem_i[p])
        pltpu.async_copy(hai.at[pl.ds(row, 1), :], idx_a.at[p], sem_i[p])

    def wait_idx(row, p):
        pltpu.make_async_copy(hii.at[pl.ds(row, 1), :], idx_i.at[p],
                              sem_i[p]).wait()
        pltpu.make_async_copy(hai.at[pl.ds(row, 1), :], idx_a.at[p],
                              sem_i[p]).wait()

    def issue_rate(row, p):
        pltpu.async_copy(hri.at[pl.ds(row, 1), :],
                         rate_buf.at[p, :, pl.ds(0, L)], sem_r[p])

    def wait_rate(row, p):
        pltpu.make_async_copy(hri.at[pl.ds(row, 1), :],
                              rate_buf.at[p, :, pl.ds(0, L)],
                              sem_r[p]).wait()

    def issue_gathers(p):
        for c, (o, n) in enumerate(I_CH):
            pltpu.async_copy(item_t.at[idx_i.at[p, 0, pl.ds(o, n)]],
                             rows_i.at[p, c, pl.ds(0, n)], sem_g[p])
        for c, (o, n) in enumerate(A_CH):
            pltpu.async_copy(attr_t.at[idx_a.at[p, 0, pl.ds(o, n)]],
                             rows_a.at[p, c, pl.ds(0, n)], sem_g[p])

    def wait_gathers(p):
        for c, (o, n) in enumerate(I_CH):
            pltpu.make_async_copy(item_t.at[idx_i.at[p, 0, pl.ds(o, n)]],
                                  rows_i.at[p, c, pl.ds(0, n)],
                                  sem_g[p]).wait()
        for c, (o, n) in enumerate(A_CH):
            pltpu.make_async_copy(attr_t.at[idx_a.at[p, 0, pl.ds(o, n)]],
                                  rows_a.at[p, c, pl.ds(0, n)],
                                  sem_g[p]).wait()

    issue_idx(base + 0, 0)
    issue_idx(base + 1, 1)
    issue_rate(base + 0, 0)
    issue_rate(base + 1, 1)

    pad = jnp.full((16,), NRATE, jnp.int32)
    for p in range(2):
        rate_buf[p, 0, pl.ds(192, 16)] = pad

    pltpu.sync_copy(rate_t, rtab)
    rt = [(rtab[rr, pl.ds(0, H)], rtab[rr, pl.ds(H, H)])
          for rr in range(NRATE)]

    pltpu.sync_copy(tii.at[pl.ds(base, BPW)], ti_idx)
    pltpu.sync_copy(tai.at[pl.ds(2 * base, 2 * BPW)], ta_idx)
    cps = [pltpu.async_copy(item_t.at[ti_idx], rows_ti, sem),
           pltpu.async_copy(attr_t.at[ta_idx.at[pl.ds(0, BPW)]],
                            rows_ta.at[0], sem),
           pltpu.async_copy(attr_t.at[ta_idx.at[pl.ds(BPW, BPW)]],
                            rows_ta.at[1], sem)]
    for c in cps:
        c.wait()

    def tgt_item_body(r, _):
        out_v[r, 0, pl.ds(0, H)] = rows_ti[r, pl.ds(0, H)]
        out_v[r, 0, pl.ds(H, H)] = rows_ti[r, pl.ds(H, H)]
        return 0
    lax.fori_loop(0, BPW, tgt_item_body, 0)

    for h in range(2):
        def tgt_attr_body(q, _, h=h):
            r = 64 * h + q
            for s in range(2):
                j = 2 * q + s
                out_v[r, 1 + s, pl.ds(0, H)] = rows_ta[h, j, pl.ds(0, H)]
                out_v[r, 1 + s, pl.ds(H, H)] = rows_ta[h, j, pl.ds(H, H)]
            return 0
        lax.fori_loop(0, 64, tgt_attr_body, 0)

    wait_idx(base + 0, 0)
    issue_gathers(0)
    wait_idx(base + 1, 1)
    issue_gathers(1)

    lane = jnp.arange(16, dtype=jnp.int32)
    rots = [((lane + sh) & 15) for sh in (8, 4, 2, 1)]
    dnums = lax.GatherDimensionNumbers(
        offset_dims=(), collapsed_slice_dims=(0,), start_index_map=(0,))

    def rot16(c, rot):
        return lax.gather(c, rot[:, None], dnums, (1,),
                          mode=lax.GatherScatterMode.PROMISE_IN_BOUNDS)

    def accum(r, p):
        acc = (zf, zf)
        for c, (o, n) in enumerate(I_CH):
            def ibody(j, a, c=c):
                return (a[0] + rows_i[p, c, j, pl.ds(0, H)],
                        a[1] + rows_i[p, c, j, pl.ds(H, H)])
            acc = lax.fori_loop(0, n, ibody, acc, unroll=4)
        out_v[r, 3, pl.ds(0, H)] = acc[0] * INV_L
        out_v[r, 3, pl.ds(H, H)] = acc[1] * INV_L

        a45 = ((zf, zf), (zf, zf))
        for c, (o, n) in enumerate(A_CH):
            def abody(j, a, c=c):
                (l4, h4), (l5, h5) = a
                return ((l4 + rows_a[p, c, 2 * j, pl.ds(0, H)],
                         h4 + rows_a[p, c, 2 * j, pl.ds(H, H)]),
                        (l5 + rows_a[p, c, 2 * j + 1, pl.ds(0, H)],
                         h5 + rows_a[p, c, 2 * j + 1, pl.ds(H, H)]))
            a45 = lax.fori_loop(0, n // 2, abody, a45, unroll=4)
        out_v[r, 4, pl.ds(0, H)] = a45[0][0] * INV_L
        out_v[r, 4, pl.ds(H, H)] = a45[0][1] * INV_L
        out_v[r, 5, pl.ds(0, H)] = a45[1][0] * INV_L
        out_v[r, 5, pl.ds(H, H)] = a45[1][1] * INV_L

        onef = jnp.ones((16,), jnp.float32)
        cnts = [zf] * NRATE
        for k in range(NKV):
            v = rate_buf[p, 0, pl.ds(16 * k, 16)]
            for rr in range(NRATE):
                cnts[rr] = cnts[rr] + jnp.where(v == rr, onef, zf)
        lo, hi = zf, zf
        for rr in range(NRATE):
            c = cnts[rr]
            for rot in rots:
                c = c + rot16(c, rot)
            lo = lo + c * rt[rr][0]
            hi = hi + c * rt[rr][1]
        out_v[r, 6, pl.ds(0, H)] = lo * INV_L
        out_v[r, 6, pl.ds(H, H)] = hi * INV_L

    def gbody(g, _):
        for p in range(2):
            r = 2 * g + p
            wait_gathers(p)

            @pl.when(r + 2 < BPW)
            def _():
                issue_idx(base + r + 2, p)

            wait_rate(base + r, p)
            accum(r, p)

            @pl.when(r + 2 < BPW)
            def _():
                issue_rate(base + r + 2, p)
                wait_idx(base + r + 2, p)
                issue_gathers(p)
        return 0

    lax.fori_loop(0, BPW // 2, gbody, 0)
    pltpu.sync_copy(out_v, out.at[pl.ds(base, BPW)])


_sc_call = functools.partial(
    pl.kernel,
    out_type=jax.ShapeDtypeStruct((B, 7, E), jnp.float32),
    mesh=plsc.VectorSubcoreMesh(core_axis_name="c", subcore_axis_name="s"),
    compiler_params=pltpu.CompilerParams(use_tc_tiling_on_sc=False),
    scratch_types=[
        pltpu.VMEM((2, 1, L), jnp.int32),
        pltpu.VMEM((2, 1, 2 * L), jnp.int32),
        pltpu.VMEM((2, 1, NKV * 16), jnp.int32),
        pltpu.VMEM((2, 2, 104, E), jnp.float32),
        pltpu.VMEM((2, 4, 104, E), jnp.float32),
        pltpu.VMEM((NRATE + 1, E), jnp.float32),
        pltpu.VMEM((BPW,), jnp.int32),
        pltpu.VMEM((2 * BPW,), jnp.int32),
        pltpu.VMEM((BPW, E), jnp.float32),
        pltpu.VMEM((2, BPW, E), jnp.float32),
        pltpu.VMEM((BPW, 7, E), jnp.float32),
        pltpu.SemaphoreType.DMA,
        pltpu.SemaphoreType.DMA,
        pltpu.SemaphoreType.DMA,
        pltpu.SemaphoreType.DMA,
        pltpu.SemaphoreType.DMA,
        pltpu.SemaphoreType.DMA,
        pltpu.SemaphoreType.DMA,
    ],
)(_body)


def kernel(hist_iid_seq, hist_aid_seq, hist_rate_seq, hist_seq_len, iid, aid,
           lb, item_table, attr_table, rating_table):
    return _sc_call(hist_iid_seq.astype(jnp.int32),
                    hist_aid_seq.astype(jnp.int32).reshape(B, 2 * L),
                    hist_rate_seq.astype(jnp.int32),
                    iid.astype(jnp.int32),
                    aid.astype(jnp.int32).reshape(2 * B),
                    item_table, attr_table, rating_table)
